# Initial kernel scaffold; baseline (speedup 1.0000x reference)
import jax
import jax.numpy as jnp
from jax.experimental import pallas as pl


def kernel(x):
    raise NotImplementedError("write your pallas kernel here")


# speedup vs baseline: 1.6472x; 1.6472x over previous
import functools
import math

import jax
import jax.numpy as jnp
from jax import lax
from jax.experimental import pallas as pl
from jax.experimental.pallas import tpu as pltpu
from jax.experimental.pallas import tpu_sc as plsc

_EMBED_DIM = 1024
_PADDING_IDX = 1


def _build_table(num_embeddings: int, embed_dim: int, padding_idx: int):
    half = embed_dim // 2
    scale = math.log(10000.0) / (half - 1)
    inv = jnp.exp(jnp.arange(half, dtype=jnp.float32) * -scale)
    pos = jnp.arange(num_embeddings, dtype=jnp.float32)
    ang = pos[:, None] * inv[None, :]
    emb = jnp.concatenate([jnp.sin(ang), jnp.cos(ang)], axis=1)
    emb = emb.at[padding_idx, :].set(0.0)
    return emb


def kernel(x):
    bsz, seq_len = x.shape
    n_rows = bsz * seq_len
    table = _build_table(_PADDING_IDX + 1 + seq_len, _EMBED_DIM, _PADDING_IDX)
    xf = x.reshape(n_rows)

    info = plsc.get_sparse_core_info()
    nc, ns, lanes = info.num_cores, info.num_subcores, info.num_lanes
    nw = nc * ns
    rows_per_w = n_rows // nw
    chunk = 64
    n_chunks = rows_per_w // chunk

    mesh = plsc.VectorSubcoreMesh(core_axis_name="c", subcore_axis_name="s")

    @functools.partial(
        pl.kernel,
        mesh=mesh,
        out_type=jax.ShapeDtypeStruct((n_rows, _EMBED_DIM), jnp.float32),
        scratch_types=[
            pltpu.VMEM((chunk,), jnp.int32),
            pltpu.VMEM((chunk,), jnp.int32),
            pltpu.VMEM((chunk, _EMBED_DIM), jnp.float32),
            pltpu.SemaphoreType.DMA,
        ],
    )
    def sc_kernel(table_hbm, x_hbm, out_hbm, xv, idxv, rows, sem):
        wid = lax.axis_index("s") * nc + lax.axis_index("c")
        wbase = wid * rows_per_w

        def body(ci, carry):
            base = wbase + ci * chunk
            tbase = lax.rem(base, seq_len)
            pltpu.sync_copy(x_hbm.at[pl.ds(base, chunk)], xv)
            for i in range(chunk // lanes):
                toks = xv[pl.ds(i * lanes, lanes)]
                seq_pos = lax.iota(jnp.int32, lanes) + (
                    tbase + i * lanes + _PADDING_IDX + 1
                )
                p = jnp.where(toks != _PADDING_IDX, seq_pos, _PADDING_IDX)
                idxv[pl.ds(i * lanes, lanes)] = p
            pltpu.async_copy(table_hbm.at[idxv], rows, sem).wait()
            pltpu.sync_copy(rows, out_hbm.at[pl.ds(base, chunk)])
            return carry

        lax.fori_loop(0, n_chunks, body, 0)

    out = sc_kernel(table, xf)
    return out.reshape(bsz, seq_len, _EMBED_DIM)


# speedup vs baseline: 1.7548x; 1.0654x over previous
import functools
import math

import jax
import jax.numpy as jnp
from jax import lax
from jax.experimental import pallas as pl
from jax.experimental.pallas import tpu as pltpu
from jax.experimental.pallas import tpu_sc as plsc

_EMBED_DIM = 1024
_PADDING_IDX = 1


def _build_table(num_embeddings: int, embed_dim: int, padding_idx: int):
    half = embed_dim // 2
    scale = math.log(10000.0) / (half - 1)
    inv = jnp.exp(jnp.arange(half, dtype=jnp.float32) * -scale)
    pos = jnp.arange(num_embeddings, dtype=jnp.float32)
    ang = pos[:, None] * inv[None, :]
    emb = jnp.concatenate([jnp.sin(ang), jnp.cos(ang)], axis=1)
    emb = emb.at[padding_idx, :].set(0.0)
    return emb


def kernel(x):
    bsz, seq_len = x.shape
    n_rows = bsz * seq_len
    table = _build_table(_PADDING_IDX + 1 + seq_len, _EMBED_DIM, _PADDING_IDX)
    xf = x.reshape(n_rows)

    info = plsc.get_sparse_core_info()
    nc, ns, lanes = info.num_cores, info.num_subcores, info.num_lanes
    nw = nc * ns
    rows_per_w = n_rows // nw
    chunk = 32
    n_chunks = rows_per_w // chunk
    nbuf = 2

    mesh = plsc.VectorSubcoreMesh(core_axis_name="c", subcore_axis_name="s")

    @functools.partial(
        pl.kernel,
        mesh=mesh,
        out_type=jax.ShapeDtypeStruct((n_rows, _EMBED_DIM), jnp.float32),
        scratch_types=[
            pltpu.VMEM((rows_per_w,), jnp.int32),
            pltpu.VMEM((n_chunks, chunk), jnp.int32),
            pltpu.VMEM((nbuf, chunk, _EMBED_DIM), jnp.float32),
            pltpu.SemaphoreType.DMA,
            pltpu.SemaphoreType.DMA,
            pltpu.SemaphoreType.DMA,
            pltpu.SemaphoreType.DMA,
        ],
    )
    def sc_kernel(table_hbm, x_hbm, out_hbm, xv, idxv, rows, sg0, sg1, ss0, ss1):
        wid = lax.axis_index("s") * nc + lax.axis_index("c")
        wbase = wid * rows_per_w
        tbase = lax.rem(wbase, seq_len)
        sg = (sg0, sg1)
        ss = (ss0, ss1)

        pltpu.sync_copy(x_hbm.at[pl.ds(wbase, rows_per_w)], xv)
        for i in range(rows_per_w // lanes):
            toks = xv[pl.ds(i * lanes, lanes)]
            seq_pos = lax.iota(jnp.int32, lanes) + (
                tbase + i * lanes + _PADDING_IDX + 1
            )
            p = jnp.where(toks != _PADDING_IDX, seq_pos, _PADDING_IDX)
            ci, j = divmod(i * lanes, chunk)
            idxv[ci, pl.ds(j, lanes)] = p

        def start_gather(ci, b):
            pltpu.async_copy(table_hbm.at[idxv.at[ci]], rows.at[b], sg[b])

        for b in range(nbuf):
            start_gather(b, b)
        for ci in range(n_chunks):
            b = ci % nbuf
            pltpu.make_async_copy(table_hbm.at[idxv.at[ci]], rows.at[b], sg[b]).wait()
            out_slice = out_hbm.at[pl.ds(wbase + ci * chunk, chunk)]
            cp = pltpu.make_async_copy(rows.at[b], out_slice, ss[b])
            cp.start()
            if ci + nbuf < n_chunks:
                cp.wait()
                start_gather(ci + nbuf, b)
            else:
                cp.wait()

    out = sc_kernel(table, xf)
    return out.reshape(bsz, seq_len, _EMBED_DIM)
